# baseline (speedup 1.0000x reference)
import math

import jax
import jax.numpy as jnp
from jax.experimental import pallas as pl
from jax.experimental.pallas import tpu as pltpu

_EPS = 1e-10
_SUMS_ROWS = 32


def _stats_body(pred_ref, inp_ref, heart_ref, out_ref, *, K, M, R):
    c = pl.program_id(1)
    mask = (heart_ref[0] == 1).astype(jnp.float32)
    pw = pred_ref[0] * mask
    rows = [mask, pw]
    for m in range(M):
        x = inp_ref[0, m:m + 1]
        rows.append(pw * x)
    for m in range(M):
        x = inp_ref[0, m:m + 1]
        rows.append(pw * (x * x))
    rows = jnp.concatenate(rows, axis=0)
    s = jnp.sum(rows, axis=1, keepdims=True)
    s = jnp.concatenate(
        [s, jnp.zeros((_SUMS_ROWS - R, 1), jnp.float32)], axis=0)

    @pl.when(c == 0)
    def _():
        out_ref[...] = jnp.zeros_like(out_ref)

    out_ref[...] += s[None]


def _loss_body(inp_ref, heart_ref, sums_ref, out_ref, *, K, M, B):
    b = pl.program_id(0)
    c = pl.program_id(1)

    nvec = sums_ref[0, 0:1, :]
    sp = sums_ref[0, 1:1 + K, :]
    d = sp + _EPS
    mus, hs = [], []
    prodvar = None
    for m in range(M):
        o1 = 1 + K + m * K
        o2 = 1 + K + (M + m) * K
        t1 = sums_ref[0, o1:o1 + K, :]
        t2 = sums_ref[0, o2:o2 + K, :]
        mu = t1 / d
        var = jnp.maximum(t2 - 2.0 * mu * t1 + mu * mu * sp, 0.0) / d + _EPS
        mus.append(mu)
        hs.append(-0.5 / var)
        prodvar = var if prodvar is None else prodvar * var
    alpha = sp / nvec
    coef = alpha * ((2.0 * math.pi) ** (-0.5 * M)) * jax.lax.rsqrt(prodvar)

    e = None
    for m in range(M):
        x = inp_ref[0, m:m + 1]
        dif = x - mus[m]
        term = hs[m] * (dif * dif)
        e = term if e is None else e + term
    sexp = jnp.sum(coef * jnp.exp(e), axis=0, keepdims=True)
    mask = (heart_ref[0] == 1).astype(jnp.float32)
    p = jnp.sum(mask * jnp.log(sexp + _EPS)).reshape(1, 1)

    @pl.when((b == 0) & (c == 0))
    def _():
        out_ref[...] = jnp.zeros_like(out_ref)

    out_ref[...] += -p / (nvec * B)


def kernel(predictions, inputs, heart):
    B, K, X, Y = predictions.shape
    M = inputs.shape[1]
    P = X * Y
    CH = 32768
    NC = P // CH
    R = 1 + K + 2 * M * K

    pred3 = predictions.reshape(B, K, P)
    inp3 = inputs.reshape(B, M, P)
    heart3 = heart.reshape(B, 1, P)

    sums = pl.pallas_call(
        lambda pr, ir, hr, orf: _stats_body(pr, ir, hr, orf, K=K, M=M, R=R),
        grid=(B, NC),
        in_specs=[
            pl.BlockSpec((1, K, CH), lambda b, c: (b, 0, c)),
            pl.BlockSpec((1, M, CH), lambda b, c: (b, 0, c)),
            pl.BlockSpec((1, 1, CH), lambda b, c: (b, 0, c)),
        ],
        out_specs=pl.BlockSpec((1, _SUMS_ROWS, 1), lambda b, c: (b, 0, 0)),
        out_shape=jax.ShapeDtypeStruct((B, _SUMS_ROWS, 1), jnp.float32),
    )(pred3, inp3, heart3)

    loss = pl.pallas_call(
        lambda ir, hr, sr, orf: _loss_body(ir, hr, sr, orf, K=K, M=M, B=B),
        grid=(B, NC),
        in_specs=[
            pl.BlockSpec((1, M, CH), lambda b, c: (b, 0, c)),
            pl.BlockSpec((1, 1, CH), lambda b, c: (b, 0, c)),
            pl.BlockSpec((1, _SUMS_ROWS, 1), lambda b, c: (b, 0, 0)),
        ],
        out_specs=pl.BlockSpec((1, 1), lambda b, c: (0, 0)),
        out_shape=jax.ShapeDtypeStruct((1, 1), jnp.float32),
    )(inp3, heart3, sums)

    return loss.reshape(())


# speedup vs baseline: 1.9933x; 1.9933x over previous
import math

import jax
import jax.numpy as jnp
from jax.experimental import pallas as pl
from jax.experimental.pallas import tpu as pltpu

_EPS = 1e-10


def _stats_body(pred_ref, inp_ref, heart_ref, coef_ref, acc_ref,
                *, K, M, B, NC, S):
    c = pl.program_id(1)
    mask = (heart_ref[0, 0] == 1).astype(jnp.float32)
    xs = [inp_ref[0, m] for m in range(M)]
    prods = [mask]
    for k in range(K):
        pw = pred_ref[0, k] * mask
        prods.append(pw)
        for m in range(M):
            prods.append(pw * xs[m])
        for m in range(M):
            prods.append(pw * (xs[m] * xs[m]))

    @pl.when(c == 0)
    def _():
        acc_ref[...] = jnp.zeros_like(acc_ref)

    for r, prod in enumerate(prods):
        acc_ref[r] += jnp.sum(prod.reshape(S // 8, 8, 128), axis=0)

    @pl.when(c == NC - 1)
    def _():
        def tot(r):
            return jnp.sum(acc_ref[r], keepdims=True)

        n = tot(0)
        G = 2 * M + 1
        for k in range(K):
            base_r = 1 + k * (1 + 2 * M)
            sp = tot(base_r)
            d = sp + _EPS
            alpha = sp / n
            prodvar = None
            for m in range(M):
                t1 = tot(base_r + 1 + m)
                t2 = tot(base_r + 1 + M + m)
                mu = t1 / d
                var = jnp.maximum(t2 - 2.0 * mu * t1 + mu * mu * sp,
                                  0.0) / d + _EPS
                coef_ref[0, k * G + m] = mu
                coef_ref[0, k * G + M + m] = -0.5 / var
                prodvar = var if prodvar is None else prodvar * var
            coef_ref[0, k * G + 2 * M] = (
                alpha * ((2.0 * math.pi) ** (-0.5 * M))
                * jax.lax.rsqrt(prodvar))
        coef_ref[0, K * G] = 1.0 / (n * B)


def _loss_body(inp_ref, heart_ref, coef_ref, out_ref, *, K, M):
    b = pl.program_id(0)
    c = pl.program_id(1)
    mask = (heart_ref[0, 0] == 1).astype(jnp.float32)
    xs = [inp_ref[0, m] for m in range(M)]
    G = 2 * M + 1
    s = None
    for k in range(K):
        e = None
        for m in range(M):
            mu = coef_ref[0, k * G + m]
            h = coef_ref[0, k * G + M + m]
            dd = xs[m] - mu
            t = h * (dd * dd)
            e = t if e is None else e + t
        term = coef_ref[0, k * G + 2 * M] * jnp.exp(e)
        s = term if s is None else s + term
    inv_nB = coef_ref[0, K * G]
    p = jnp.sum(mask * jnp.log(s + _EPS), keepdims=True)

    @pl.when((b == 0) & (c == 0))
    def _():
        out_ref[...] = jnp.zeros_like(out_ref)

    out_ref[...] += -(p * inv_nB)


def kernel(predictions, inputs, heart):
    B, K, X, Y = predictions.shape
    M = inputs.shape[1]
    P = X * Y
    S = 256
    NC = P // (S * 128)
    R = 1 + K * (1 + 2 * M)
    NCOEF = K * (2 * M + 1) + 1

    pred4 = predictions.reshape(B, K, P // 128, 128)
    inp4 = inputs.reshape(B, M, P // 128, 128)
    heart4 = heart.reshape(B, 1, P // 128, 128)

    coef = pl.pallas_call(
        lambda pr, ir, hr, cr, ar: _stats_body(
            pr, ir, hr, cr, ar, K=K, M=M, B=B, NC=NC, S=S),
        grid=(B, NC),
        in_specs=[
            pl.BlockSpec((1, K, S, 128), lambda b, c: (b, 0, c, 0)),
            pl.BlockSpec((1, M, S, 128), lambda b, c: (b, 0, c, 0)),
            pl.BlockSpec((1, 1, S, 128), lambda b, c: (b, 0, c, 0)),
        ],
        out_specs=pl.BlockSpec((1, NCOEF, 1, 1), lambda b, c: (b, 0, 0, 0)),
        out_shape=jax.ShapeDtypeStruct((B, NCOEF, 1, 1), jnp.float32),
        scratch_shapes=[pltpu.VMEM((R, 8, 128), jnp.float32)],
    )(pred4, inp4, heart4)

    loss = pl.pallas_call(
        lambda ir, hr, cr, orf: _loss_body(ir, hr, cr, orf, K=K, M=M),
        grid=(B, NC),
        in_specs=[
            pl.BlockSpec((1, M, S, 128), lambda b, c: (b, 0, c, 0)),
            pl.BlockSpec((1, 1, S, 128), lambda b, c: (b, 0, c, 0)),
            pl.BlockSpec((1, NCOEF, 1, 1), lambda b, c: (b, 0, 0, 0)),
        ],
        out_specs=pl.BlockSpec((1, 1), lambda b, c: (0, 0)),
        out_shape=jax.ShapeDtypeStruct((1, 1), jnp.float32),
    )(inp4, heart4, coef)

    return loss.reshape(())


# speedup vs baseline: 3.1369x; 1.5737x over previous
import math

import jax
import jax.numpy as jnp
from jax.experimental import pallas as pl
from jax.experimental.pallas import tpu as pltpu

_EPS = 1e-10


def _stats_body(pred_ref, inp_ref, heart_ref, coef_ref, acc_ref,
                *, K, M, B, NC, S, Y):
    c = pl.program_id(1)
    mask = (heart_ref[0, 0] == 1).astype(jnp.float32)
    xs = [inp_ref[0, m] for m in range(M)]
    prods = [mask]
    for k in range(K):
        pw = pred_ref[0, k] * mask
        prods.append(pw)
        for m in range(M):
            prods.append(pw * xs[m])
        for m in range(M):
            prods.append(pw * (xs[m] * xs[m]))

    @pl.when(c == 0)
    def _():
        acc_ref[...] = jnp.zeros_like(acc_ref)

    for r, prod in enumerate(prods):
        acc_ref[r] += jnp.sum(prod.reshape(S // 8, 8, Y), axis=0)

    @pl.when(c == NC - 1)
    def _():
        def tot(r):
            return jnp.sum(acc_ref[r], keepdims=True)

        n = tot(0)
        G = 2 * M + 1
        for k in range(K):
            base_r = 1 + k * (1 + 2 * M)
            sp = tot(base_r)
            d = sp + _EPS
            alpha = sp / n
            prodvar = None
            for m in range(M):
                t1 = tot(base_r + 1 + m)
                t2 = tot(base_r + 1 + M + m)
                mu = t1 / d
                var = jnp.maximum(t2 - 2.0 * mu * t1 + mu * mu * sp,
                                  0.0) / d + _EPS
                coef_ref[0, k * G + m] = mu
                coef_ref[0, k * G + M + m] = -0.5 / var
                prodvar = var if prodvar is None else prodvar * var
            coef_ref[0, k * G + 2 * M] = (
                alpha * ((2.0 * math.pi) ** (-0.5 * M))
                * jax.lax.rsqrt(prodvar))
        coef_ref[0, K * G] = 1.0 / (n * B)


def _loss_body(inp_ref, heart_ref, coef_ref, out_ref, *, K, M):
    b = pl.program_id(0)
    c = pl.program_id(1)
    mask = (heart_ref[0, 0] == 1).astype(jnp.float32)
    xs = [inp_ref[0, m] for m in range(M)]
    G = 2 * M + 1
    s = None
    for k in range(K):
        e = None
        for m in range(M):
            mu = coef_ref[0, k * G + m]
            h = coef_ref[0, k * G + M + m]
            dd = xs[m] - mu
            t = h * (dd * dd)
            e = t if e is None else e + t
        term = coef_ref[0, k * G + 2 * M] * jnp.exp(e)
        s = term if s is None else s + term
    inv_nB = coef_ref[0, K * G]
    p = jnp.sum(mask * jnp.log(s + _EPS), keepdims=True)

    @pl.when((b == 0) & (c == 0))
    def _():
        out_ref[...] = jnp.zeros_like(out_ref)

    out_ref[...] += -(p * inv_nB)


def kernel(predictions, inputs, heart):
    B, K, X, Y = predictions.shape
    M = inputs.shape[1]
    S = 64
    NC = X // S
    R = 1 + K * (1 + 2 * M)
    NCOEF = K * (2 * M + 1) + 1

    coef = pl.pallas_call(
        lambda pr, ir, hr, cr, ar: _stats_body(
            pr, ir, hr, cr, ar, K=K, M=M, B=B, NC=NC, S=S, Y=Y),
        grid=(B, NC),
        in_specs=[
            pl.BlockSpec((1, K, S, Y), lambda b, c: (b, 0, c, 0)),
            pl.BlockSpec((1, M, S, Y), lambda b, c: (b, 0, c, 0)),
            pl.BlockSpec((1, 1, S, Y), lambda b, c: (b, 0, c, 0)),
        ],
        out_specs=pl.BlockSpec((1, NCOEF, 1, 1), lambda b, c: (b, 0, 0, 0)),
        out_shape=jax.ShapeDtypeStruct((B, NCOEF, 1, 1), jnp.float32),
        scratch_shapes=[pltpu.VMEM((R, 8, Y), jnp.float32)],
    )(predictions, inputs, heart)

    loss = pl.pallas_call(
        lambda ir, hr, cr, orf: _loss_body(ir, hr, cr, orf, K=K, M=M),
        grid=(B, NC),
        in_specs=[
            pl.BlockSpec((1, M, S, Y), lambda b, c: (b, 0, c, 0)),
            pl.BlockSpec((1, 1, S, Y), lambda b, c: (b, 0, c, 0)),
            pl.BlockSpec((1, NCOEF, 1, 1), lambda b, c: (b, 0, 0, 0)),
        ],
        out_specs=pl.BlockSpec((1, 1), lambda b, c: (0, 0)),
        out_shape=jax.ShapeDtypeStruct((1, 1), jnp.float32),
    )(inputs, heart, coef)

    return loss.reshape(())


# speedup vs baseline: 4.5111x; 1.4381x over previous
import math

import jax
import jax.numpy as jnp
from jax.experimental import pallas as pl
from jax.experimental.pallas import tpu as pltpu

_EPS = 1e-10


def _stats_body(pred_ref, inp_ref, heart_ref, coef_ref, acc_ref,
                *, K, M, B, NC, S, Y):
    c = pl.program_id(1)
    mask = (heart_ref[0, 0] == 1).astype(jnp.float32)
    xs = [inp_ref[0, m] for m in range(M)]
    prods = [mask]
    for k in range(K):
        pw = pred_ref[0, k] * mask
        prods.append(pw)
        for m in range(M):
            prods.append(pw * xs[m])
        for m in range(M):
            prods.append(pw * (xs[m] * xs[m]))

    @pl.when(c == 0)
    def _():
        acc_ref[...] = jnp.zeros_like(acc_ref)

    for r, prod in enumerate(prods):
        acc_ref[r] += jnp.sum(prod.reshape(S // 8, 8, Y), axis=0)

    @pl.when(c == NC - 1)
    def _():
        def tot(r):
            return jnp.sum(acc_ref[r], keepdims=True)

        n = tot(0)
        G = 2 * M + 1
        for k in range(K):
            base_r = 1 + k * (1 + 2 * M)
            sp = tot(base_r)
            d = sp + _EPS
            alpha = sp / n
            prodvar = None
            for m in range(M):
                t1 = tot(base_r + 1 + m)
                t2 = tot(base_r + 1 + M + m)
                mu = t1 / d
                var = jnp.maximum(t2 - 2.0 * mu * t1 + mu * mu * sp,
                                  0.0) / d + _EPS
                coef_ref[0, k * G + m] = mu
                coef_ref[0, k * G + M + m] = -0.5 / var
                prodvar = var if prodvar is None else prodvar * var
            coef_ref[0, k * G + 2 * M] = (
                alpha * ((2.0 * math.pi) ** (-0.5 * M))
                * jax.lax.rsqrt(prodvar))
        coef_ref[0, K * G] = 1.0 / (n * B)


def _loss_body(inp_ref, heart_ref, coef_ref, out_ref, *, K, M):
    b = pl.program_id(0)
    c = pl.program_id(1)
    mask = (heart_ref[0, 0] == 1).astype(jnp.float32)
    xs = [inp_ref[0, m] for m in range(M)]
    G = 2 * M + 1
    s = None
    for k in range(K):
        e = None
        for m in range(M):
            mu = coef_ref[0, k * G + m]
            h = coef_ref[0, k * G + M + m]
            dd = xs[m] - mu
            t = h * (dd * dd)
            e = t if e is None else e + t
        term = coef_ref[0, k * G + 2 * M] * jnp.exp(e)
        s = term if s is None else s + term
    inv_nB = coef_ref[0, K * G]
    p = jnp.sum(mask * jnp.log(s + _EPS), keepdims=True)

    @pl.when((b == 0) & (c == 0))
    def _():
        out_ref[...] = jnp.zeros_like(out_ref)

    out_ref[...] += -(p * inv_nB)


def kernel(predictions, inputs, heart):
    B, K, X, Y = predictions.shape
    M = inputs.shape[1]
    S = 128
    NC = X // S
    R = 1 + K * (1 + 2 * M)
    NCOEF = K * (2 * M + 1) + 1

    coef = pl.pallas_call(
        lambda pr, ir, hr, cr, ar: _stats_body(
            pr, ir, hr, cr, ar, K=K, M=M, B=B, NC=NC, S=S, Y=Y),
        grid=(B, NC),
        in_specs=[
            pl.BlockSpec((1, K, S, Y), lambda b, c: (b, 0, c, 0)),
            pl.BlockSpec((1, M, S, Y), lambda b, c: (b, 0, c, 0)),
            pl.BlockSpec((1, 1, S, Y), lambda b, c: (b, 0, c, 0)),
        ],
        out_specs=pl.BlockSpec((1, NCOEF, 1, 1), lambda b, c: (b, 0, 0, 0)),
        out_shape=jax.ShapeDtypeStruct((B, NCOEF, 1, 1), jnp.float32),
        scratch_shapes=[pltpu.VMEM((R, 8, Y), jnp.float32)],
    )(predictions, inputs, heart)

    loss = pl.pallas_call(
        lambda ir, hr, cr, orf: _loss_body(ir, hr, cr, orf, K=K, M=M),
        grid=(B, NC),
        in_specs=[
            pl.BlockSpec((1, M, S, Y), lambda b, c: (b, 0, c, 0)),
            pl.BlockSpec((1, 1, S, Y), lambda b, c: (b, 0, c, 0)),
            pl.BlockSpec((1, NCOEF, 1, 1), lambda b, c: (b, 0, 0, 0)),
        ],
        out_specs=pl.BlockSpec((1, 1), lambda b, c: (0, 0)),
        out_shape=jax.ShapeDtypeStruct((1, 1), jnp.float32),
    )(inputs, heart, coef)

    return loss.reshape(())


# speedup vs baseline: 5.3704x; 1.1905x over previous
import math

import jax
import jax.numpy as jnp
from jax.experimental import pallas as pl
from jax.experimental.pallas import tpu as pltpu

_EPS = 1e-10


def _stats_body(pred_ref, inp_ref, heart_ref, coef_ref, acc_ref,
                *, K, M, B, NC, S, Y):
    c = pl.program_id(1)
    mask = (heart_ref[0, 0] == 1).astype(jnp.float32)
    xs = [inp_ref[0, m] for m in range(M)]
    prods = [mask]
    for k in range(K):
        pw = pred_ref[0, k] * mask
        prods.append(pw)
        for m in range(M):
            prods.append(pw * xs[m])
        for m in range(M):
            prods.append(pw * (xs[m] * xs[m]))

    @pl.when(c == 0)
    def _():
        acc_ref[...] = jnp.zeros_like(acc_ref)

    for r, prod in enumerate(prods):
        acc_ref[r] += jnp.sum(prod.reshape(S // 8, 8, Y), axis=0)

    @pl.when(c == NC - 1)
    def _():
        def tot(r):
            return jnp.sum(acc_ref[r], keepdims=True)

        n = tot(0)
        G = 2 * M + 1
        for k in range(K):
            base_r = 1 + k * (1 + 2 * M)
            sp = tot(base_r)
            d = sp + _EPS
            alpha = sp / n
            prodvar = None
            for m in range(M):
                t1 = tot(base_r + 1 + m)
                t2 = tot(base_r + 1 + M + m)
                mu = t1 / d
                var = jnp.maximum(t2 - 2.0 * mu * t1 + mu * mu * sp,
                                  0.0) / d + _EPS
                coef_ref[0, k * G + m] = mu
                coef_ref[0, k * G + M + m] = -0.5 / var
                prodvar = var if prodvar is None else prodvar * var
            coef_ref[0, k * G + 2 * M] = (
                alpha * ((2.0 * math.pi) ** (-0.5 * M))
                * jax.lax.rsqrt(prodvar))
        coef_ref[0, K * G] = 1.0 / (n * B)


def _loss_body(inp_ref, heart_ref, coef_ref, out_ref, *, K, M):
    b = pl.program_id(0)
    c = pl.program_id(1)
    mask = (heart_ref[0, 0] == 1).astype(jnp.float32)
    xs = [inp_ref[0, m] for m in range(M)]
    G = 2 * M + 1
    s = None
    for k in range(K):
        e = None
        for m in range(M):
            mu = coef_ref[0, k * G + m]
            h = coef_ref[0, k * G + M + m]
            dd = xs[m] - mu
            t = h * (dd * dd)
            e = t if e is None else e + t
        term = coef_ref[0, k * G + 2 * M] * jnp.exp(e)
        s = term if s is None else s + term
    inv_nB = coef_ref[0, K * G]
    p = jnp.sum(mask * jnp.log(s + _EPS), keepdims=True)

    @pl.when((b == 0) & (c == 0))
    def _():
        out_ref[...] = jnp.zeros_like(out_ref)

    out_ref[...] += -(p * inv_nB)


def kernel(predictions, inputs, heart):
    B, K, X, Y = predictions.shape
    M = inputs.shape[1]
    S = 256
    NC = X // S
    R = 1 + K * (1 + 2 * M)
    NCOEF = K * (2 * M + 1) + 1

    coef = pl.pallas_call(
        lambda pr, ir, hr, cr, ar: _stats_body(
            pr, ir, hr, cr, ar, K=K, M=M, B=B, NC=NC, S=S, Y=Y),
        grid=(B, NC),
        in_specs=[
            pl.BlockSpec((1, K, S, Y), lambda b, c: (b, 0, c, 0)),
            pl.BlockSpec((1, M, S, Y), lambda b, c: (b, 0, c, 0)),
            pl.BlockSpec((1, 1, S, Y), lambda b, c: (b, 0, c, 0)),
        ],
        out_specs=pl.BlockSpec((1, NCOEF, 1, 1), lambda b, c: (b, 0, 0, 0)),
        out_shape=jax.ShapeDtypeStruct((B, NCOEF, 1, 1), jnp.float32),
        scratch_shapes=[pltpu.VMEM((R, 8, Y), jnp.float32)],
    )(predictions, inputs, heart)

    loss = pl.pallas_call(
        lambda ir, hr, cr, orf: _loss_body(ir, hr, cr, orf, K=K, M=M),
        grid=(B, NC),
        in_specs=[
            pl.BlockSpec((1, M, S, Y), lambda b, c: (b, 0, c, 0)),
            pl.BlockSpec((1, 1, S, Y), lambda b, c: (b, 0, c, 0)),
            pl.BlockSpec((1, NCOEF, 1, 1), lambda b, c: (b, 0, 0, 0)),
        ],
        out_specs=pl.BlockSpec((1, 1), lambda b, c: (0, 0)),
        out_shape=jax.ShapeDtypeStruct((1, 1), jnp.float32),
    )(inputs, heart, coef)

    return loss.reshape(())
